# baseline (device time: 10945 ns/iter reference)
import jax
import jax.numpy as jnp
from jax import lax
from jax.experimental import pallas as pl
from jax.experimental.pallas import tpu as pltpu

N_DEV = 32
KW = 4
HALO = KW - 1
N_CHUNKS = 4


def kernel(x, k):
    b, s, c = x.shape
    cs = s // N_CHUNKS

    def body(x_any, x_blk, k_ref, out_ref,
             halo_ref, tail0_ref, carry_ref,
             send_sem, recv_sem, tail0_sem, ack_sem):
        i = pl.program_id(0)
        my = lax.axis_index("i")
        has_left = my > 0
        has_right = my < N_DEV - 1

        barrier_sem = pltpu.get_barrier_semaphore()

        rdma = pltpu.make_async_remote_copy(
            src_ref=x_any.at[:, pl.ds(s - HALO, HALO), :],
            dst_ref=halo_ref,
            send_sem=send_sem,
            recv_sem=recv_sem,
            device_id=((my + 1) % N_DEV,),
            device_id_type=pl.DeviceIdType.MESH,
        )
        tail0_dma = pltpu.make_async_copy(
            x_any.at[:, pl.ds(cs - HALO, HALO), :], tail0_ref, tail0_sem,
        )

        @pl.when(i == 0)
        def _():
            tail0_dma.start()

            @pl.when(has_left)
            def _():
                pl.semaphore_signal(barrier_sem, inc=1, device_id=(my - 1,),
                                    device_id_type=pl.DeviceIdType.MESH)

            @pl.when(has_right)
            def _():
                pl.semaphore_signal(barrier_sem, inc=1, device_id=(my + 1,),
                                    device_id_type=pl.DeviceIdType.MESH)

            n_nbrs = has_left.astype(jnp.int32) + has_right.astype(jnp.int32)
            pl.semaphore_wait(barrier_sem, n_nbrs)

            @pl.when(has_right)
            def _():
                rdma.start()

            tail0_dma.wait()
            carry_ref[...] = tail0_ref[...].astype(jnp.bfloat16)

        @pl.when(i == N_CHUNKS - 1)
        def _():
            @pl.when(has_left)
            def _():
                rdma.wait_recv()
                carry_ref[...] = halo_ref[...].astype(jnp.bfloat16)
                pl.semaphore_signal(ack_sem, inc=1, device_id=(my - 1,),
                                    device_id_type=pl.DeviceIdType.MESH)

            @pl.when(jnp.logical_not(has_left))
            def _():
                carry_ref[...] = jnp.zeros((b, HALO, c), jnp.bfloat16)

        xv = x_blk[...].astype(jnp.bfloat16)
        kv = k_ref[...].astype(jnp.bfloat16)
        pad = jnp.concatenate([carry_ref[...], xv], axis=1)
        out = jnp.zeros((b, cs, c), jnp.bfloat16)
        for t in range(KW):
            out = out + pad[:, t:t + cs, :] * kv[t][None, None, :]
        out_ref[...] = out * jax.nn.sigmoid(out)

        carry_ref[...] = xv[:, cs - HALO:, :]

        @pl.when(i == N_CHUNKS - 1)
        def _():
            @pl.when(has_right)
            def _():
                rdma.wait_send()
                pl.semaphore_wait(ack_sem, 1)

    perm = lambda j: (j + 1) % N_CHUNKS

    return pl.pallas_call(
        body,
        grid=(N_CHUNKS,),
        out_shape=jax.ShapeDtypeStruct((b, s, c), jnp.bfloat16),
        in_specs=[
            pl.BlockSpec(memory_space=pl.ANY),
            pl.BlockSpec((b, cs, c), lambda j: (0, perm(j), 0)),
            pl.BlockSpec((KW, c), lambda j: (0, 0)),
        ],
        out_specs=pl.BlockSpec((b, cs, c), lambda j: (0, perm(j), 0)),
        scratch_shapes=[
            pltpu.VMEM((b, HALO, c), x.dtype),
            pltpu.VMEM((b, HALO, c), x.dtype),
            pltpu.VMEM((b, HALO, c), jnp.bfloat16),
            pltpu.SemaphoreType.DMA,
            pltpu.SemaphoreType.DMA,
            pltpu.SemaphoreType.DMA,
            pltpu.SemaphoreType.REGULAR,
        ],
        compiler_params=pltpu.CompilerParams(
            collective_id=0,
            dimension_semantics=("arbitrary",),
        ),
    )(x, x, k)
